# baseline (speedup 1.0000x reference)
import functools

import jax
import jax.numpy as jnp
from jax import lax
from jax.experimental import pallas as pl
from jax.experimental.pallas import tpu as pltpu
from jax.experimental.pallas import tpu_sc as plsc

_ROWS = 1024

_NC = 2
_NS = 16
_NW = _NC * _NS
_NDS = 4


_BPG = 8


def _argmin_body(xt_ref, cb_ref, cbsq_ref, idx_ref):
    cb = cb_ref[...]
    t_ = xt_ref.shape[2]
    for j in range(_BPG):
        xt = xt_ref[j]
        mm = lax.dot_general(cb, xt, (((1,), (0,)), ((), ())),
                             preferred_element_type=jnp.float32)
        s = xt * xt
        w = s.shape[0]
        while w > 1:
            w //= 2
            s = s[:w] + s[w:]
        d = (cbsq_ref[...] + s) - 2.0 * mm
        k = d.shape[0]
        min_d = jnp.min(d, axis=0, keepdims=True)
        iota = lax.broadcasted_iota(jnp.int32, d.shape, 0)
        idx = jnp.min(jnp.where(d == min_d, iota, jnp.int32(k)), axis=0)
        idx_ref[j] = idx


def _argmin_call(xtt, codebook, cbsq, b0, nbat):
    _, d_, t_ = xtt.shape
    k = codebook.shape[0]
    b0g = b0 // _BPG
    return pl.pallas_call(
        _argmin_body,
        grid=(nbat // _BPG,),
        in_specs=[
            pl.BlockSpec((_BPG, d_, t_), lambda i: (i + b0g, 0, 0)),
            pl.BlockSpec((k, d_), lambda i: (0, 0)),
            pl.BlockSpec((k, 1), lambda i: (0, 0)),
        ],
        out_specs=pl.BlockSpec((_BPG, t_), lambda i: (i, 0)),
        out_shape=jax.ShapeDtypeStruct((nbat, t_), jnp.int32),
    )(xtt, codebook, cbsq)


@functools.lru_cache(maxsize=None)
def _make_gather(nb, t_, d_, k):
    n_bg = _NW // _NDS
    b_per_w = nb // n_bg
    n_per_w = b_per_w * t_
    d_sl = d_ // _NDS
    groups = t_ // 16
    mesh = plsc.VectorSubcoreMesh(core_axis_name="c", subcore_axis_name="s")

    @functools.partial(
        pl.kernel, mesh=mesh,
        compiler_params=pltpu.CompilerParams(use_tc_tiling_on_sc=False,
                                             needs_layout_passes=False),
        out_type=jax.ShapeDtypeStruct((nb, d_, t_), jnp.float32),
        scratch_types=[
            pltpu.VMEM((d_sl, k), jnp.float32),
            pltpu.VMEM((n_per_w,), jnp.int32),
            pltpu.VMEM((d_sl, t_), jnp.float32),
        ],
    )
    def gk(cbt_hbm, idx_hbm, out_hbm, cbt_v, idx_v, zqt_v):
        wid = lax.axis_index("s") * _NC + lax.axis_index("c")
        bg = wid // _NDS
        ds0 = (wid % _NDS) * d_sl
        pltpu.sync_copy(cbt_hbm.at[pl.ds(ds0, d_sl)], cbt_v)
        pltpu.sync_copy(idx_hbm.at[pl.ds(bg * n_per_w, n_per_w)], idx_v)

        for bb in range(b_per_w):
            @plsc.parallel_loop(0, groups)
            def per_group(g, bb=bb):
                iv = idx_v[pl.ds(bb * t_ + g * 16, 16)]
                for dd in range(d_sl):
                    row_sel = jnp.full((16,), dd, dtype=jnp.int32)
                    vals = plsc.load_gather(cbt_v, [row_sel, iv])
                    zqt_v[dd, pl.ds(g * 16, 16)] = vals

            b = bg * b_per_w + bb
            pltpu.sync_copy(zqt_v, out_hbm.at[b, pl.ds(ds0, d_sl)])

    return gk


def kernel(z_e_x, codebook):
    nb, t_, d_ = z_e_x.shape
    k = codebook.shape[0]
    cbsq = jnp.sum(codebook ** 2, axis=1)[:, None]
    cbt = codebook.T
    xtt = jnp.transpose(z_e_x, (0, 2, 1))
    qb = nb // 4
    gather = _make_gather(qb, t_, d_, k)
    idxs = [_argmin_call(xtt, codebook, cbsq, r * qb, qb).reshape(-1)
            for r in range(4)]
    zqts = [gather(cbt, ix) for ix in idxs]
    zqt = jnp.concatenate(zqts, axis=0)
    indices = jnp.concatenate(idxs)
    z_q = jnp.transpose(zqt, (0, 2, 1))
    return (z_q, z_q, indices)


# speedup vs baseline: 1.0298x; 1.0298x over previous
import functools

import jax
import jax.numpy as jnp
from jax import lax
from jax.experimental import pallas as pl
from jax.experimental.pallas import tpu as pltpu
from jax.experimental.pallas import tpu_sc as plsc

_ROWS = 1024

_NC = 2
_NS = 16
_NW = _NC * _NS
_NDS = 4


_BPG = 8


def _argmin_body(xt_ref, cb_ref, cbsq_ref, idx_ref):
    cb = cb_ref[...]
    t_ = xt_ref.shape[2]
    for j in range(_BPG):
        xt = xt_ref[j]
        mm = lax.dot_general(cb, xt, (((1,), (0,)), ((), ())),
                             preferred_element_type=jnp.float32)
        s = xt * xt
        w = s.shape[0]
        while w > 1:
            w //= 2
            s = s[:w] + s[w:]
        d = (cbsq_ref[...] + s) - 2.0 * mm
        k = d.shape[0]
        min_d = jnp.min(d, axis=0, keepdims=True)
        iota = lax.broadcasted_iota(jnp.int32, d.shape, 0)
        idx = jnp.min(jnp.where(d == min_d, iota, jnp.int32(k)), axis=0)
        idx_ref[j] = idx


def _argmin_call(xtt, codebook, cbsq, b0, nbat):
    _, d_, t_ = xtt.shape
    k = codebook.shape[0]
    b0g = b0 // _BPG
    return pl.pallas_call(
        _argmin_body,
        grid=(nbat // _BPG,),
        in_specs=[
            pl.BlockSpec((_BPG, d_, t_), lambda i: (i + b0g, 0, 0)),
            pl.BlockSpec((k, d_), lambda i: (0, 0)),
            pl.BlockSpec((k, 1), lambda i: (0, 0)),
        ],
        out_specs=pl.BlockSpec((_BPG, t_), lambda i: (i, 0)),
        out_shape=jax.ShapeDtypeStruct((nbat, t_), jnp.int32),
    )(xtt, codebook, cbsq)


@functools.lru_cache(maxsize=None)
def _make_gather(nb, t_, d_, k):
    n_bg = _NW // _NDS
    b_per_w = nb // n_bg
    n_per_w = b_per_w * t_
    d_sl = d_ // _NDS
    groups = t_ // 16
    mesh = plsc.VectorSubcoreMesh(core_axis_name="c", subcore_axis_name="s")

    @functools.partial(
        pl.kernel, mesh=mesh,
        compiler_params=pltpu.CompilerParams(use_tc_tiling_on_sc=False,
                                             needs_layout_passes=False),
        out_type=jax.ShapeDtypeStruct((nb, d_, t_), jnp.float32),
        scratch_types=[
            pltpu.VMEM((d_sl, k), jnp.float32),
            pltpu.VMEM((n_per_w,), jnp.int32),
            pltpu.VMEM((d_sl, t_), jnp.float32),
        ],
    )
    def gk(cbt_hbm, idx_hbm, out_hbm, cbt_v, idx_v, zqt_v):
        wid = lax.axis_index("s") * _NC + lax.axis_index("c")
        bg = wid // _NDS
        ds0 = (wid % _NDS) * d_sl
        pltpu.sync_copy(cbt_hbm.at[pl.ds(ds0, d_sl)], cbt_v)
        pltpu.sync_copy(idx_hbm.at[pl.ds(bg * n_per_w, n_per_w)], idx_v)

        for bb in range(b_per_w):
            @plsc.parallel_loop(0, groups)
            def per_group(g, bb=bb):
                iv = idx_v[pl.ds(bb * t_ + g * 16, 16)]
                for dd in range(d_sl):
                    row_sel = jnp.full((16,), dd, dtype=jnp.int32)
                    vals = plsc.load_gather(cbt_v, [row_sel, iv])
                    zqt_v[dd, pl.ds(g * 16, 16)] = vals

            b = bg * b_per_w + bb
            pltpu.sync_copy(zqt_v, out_hbm.at[b, pl.ds(ds0, d_sl)])

    return gk


def kernel(z_e_x, codebook):
    nb, t_, d_ = z_e_x.shape
    k = codebook.shape[0]
    cbsq = jnp.sum(codebook ** 2, axis=1)[:, None]
    cbt = codebook.T
    xtt = jnp.transpose(z_e_x, (0, 2, 1))
    half_b = nb // 2
    gather = _make_gather(half_b, t_, d_, k)
    idx0 = _argmin_call(xtt, codebook, cbsq, 0, half_b).reshape(-1)
    idx1 = _argmin_call(xtt, codebook, cbsq, half_b, half_b).reshape(-1)
    zqt0 = gather(cbt, idx0)
    zqt1 = gather(cbt, idx1)
    zqt = jnp.concatenate([zqt0, zqt1], axis=0)
    indices = jnp.concatenate([idx0, idx1])
    z_q = jnp.transpose(zqt, (0, 2, 1))
    return (z_q, z_q, indices)


# speedup vs baseline: 1.0306x; 1.0008x over previous
import functools

import jax
import jax.numpy as jnp
from jax import lax
from jax.experimental import pallas as pl
from jax.experimental.pallas import tpu as pltpu
from jax.experimental.pallas import tpu_sc as plsc

_NC = 2
_NS = 16
_NW = _NC * _NS
_NDS = 4

_BPG = 8


def _argmin_body(xt_ref, cb_ref, cbsq_ref, idx_ref):
    cb = cb_ref[...]
    t_ = xt_ref.shape[2]
    for j in range(_BPG):
        xt = xt_ref[j]
        mm = lax.dot_general(cb, xt, (((1,), (0,)), ((), ())),
                             preferred_element_type=jnp.float32)
        s = xt * xt
        w = s.shape[0]
        while w > 1:
            w //= 2
            s = s[:w] + s[w:]
        d = (cbsq_ref[...] + s) - 2.0 * mm
        k = d.shape[0]
        min_d = jnp.min(d, axis=0, keepdims=True)
        iota = lax.broadcasted_iota(jnp.int32, d.shape, 0)
        idx = jnp.min(jnp.where(d == min_d, iota, jnp.int32(k)), axis=0)
        idx_ref[j] = idx


def _argmin_call(xtt, codebook, cbsq, b0, nbat):
    _, d_, t_ = xtt.shape
    k = codebook.shape[0]
    b0g = b0 // _BPG
    return pl.pallas_call(
        _argmin_body,
        grid=(nbat // _BPG,),
        in_specs=[
            pl.BlockSpec((_BPG, d_, t_), lambda i: (i + b0g, 0, 0)),
            pl.BlockSpec((k, d_), lambda i: (0, 0)),
            pl.BlockSpec((k, 1), lambda i: (0, 0)),
        ],
        out_specs=pl.BlockSpec((_BPG, t_), lambda i: (i, 0)),
        out_shape=jax.ShapeDtypeStruct((nbat, t_), jnp.int32),
    )(xtt, codebook, cbsq)


@functools.lru_cache(maxsize=None)
def _make_gather(nb, t_, d_, k):
    n_bg = _NW // _NDS
    b_per_w = nb // n_bg
    n_per_w = b_per_w * t_
    d_sl = d_ // _NDS
    groups = t_ // 16
    mesh = plsc.VectorSubcoreMesh(core_axis_name="c", subcore_axis_name="s")

    @functools.partial(
        pl.kernel, mesh=mesh,
        compiler_params=pltpu.CompilerParams(use_tc_tiling_on_sc=False,
                                             needs_layout_passes=False),
        out_type=jax.ShapeDtypeStruct((nb, d_, t_), jnp.float32),
        scratch_types=[
            pltpu.VMEM((d_sl, k), jnp.float32),
            pltpu.VMEM((n_per_w,), jnp.int32),
            pltpu.VMEM((d_sl, t_), jnp.float32),
        ],
    )
    def gk(cbt_hbm, idx_hbm, out_hbm, cbt_v, idx_v, zqt_v):
        wid = lax.axis_index("s") * _NC + lax.axis_index("c")
        bg = wid // _NDS
        ds0 = (wid % _NDS) * d_sl
        pltpu.sync_copy(cbt_hbm.at[pl.ds(ds0, d_sl)], cbt_v)
        pltpu.sync_copy(idx_hbm.at[pl.ds(bg * n_per_w, n_per_w)], idx_v)

        for bb in range(b_per_w):
            @plsc.parallel_loop(0, groups)
            def per_group(g, bb=bb):
                iv = idx_v[pl.ds(bb * t_ + g * 16, 16)]
                for dd in range(d_sl):
                    row_sel = jnp.full((16,), dd, dtype=jnp.int32)
                    vals = plsc.load_gather(cbt_v, [row_sel, iv])
                    zqt_v[dd, pl.ds(g * 16, 16)] = vals

            b = bg * b_per_w + bb
            pltpu.sync_copy(zqt_v, out_hbm.at[b, pl.ds(ds0, d_sl)])

    return gk


def kernel(z_e_x, codebook):
    nb, t_, d_ = z_e_x.shape
    k = codebook.shape[0]
    cbsq = jnp.sum(codebook ** 2, axis=1)[:, None]
    cbt = codebook.T
    xtt = jnp.transpose(z_e_x, (0, 2, 1))
    half_b = nb // 2
    gather = _make_gather(half_b, t_, d_, k)
    idx0 = _argmin_call(xtt, codebook, cbsq, 0, half_b).reshape(-1)
    idx1 = _argmin_call(xtt, codebook, cbsq, half_b, half_b).reshape(-1)
    zqt0 = gather(cbt, idx0)
    zqt1 = gather(cbt, idx1)
    zqt = jnp.concatenate([zqt0, zqt1], axis=0)
    indices = jnp.concatenate([idx0, idx1])
    z_q = jnp.transpose(zqt, (0, 2, 1))
    return (z_q, z_q, indices)
